# baseline (speedup 1.0000x reference)
import functools

import jax
import jax.numpy as jnp
from jax import lax
from jax.experimental import pallas as pl
from jax.experimental.pallas import tpu as pltpu
from jax.experimental.pallas import tpu_sc as plsc

HIDDEN = 1024
FFN = 2048
E = 16
TOPK = 2
TOK = 128
FB = 1024
NF = FFN // FB

_N_ASSIGN = TOK * TOPK


def _router_kernel(te_hbm, tw_hbm, w_hbm, te_v, tw_v, w_v):
    cid = lax.axis_index("c")
    sid = lax.axis_index("s")
    wid = sid * 2 + cid

    @pl.when(wid == 0)
    def _():
        pltpu.sync_copy(te_hbm, te_v)
        pltpu.sync_copy(tw_hbm, tw_v)
        zero = jnp.zeros((16,), jnp.float32)
        for p in range(TOK // 16):
            te0 = te_v[pl.ds(p * 16, 16)]
            tw0 = tw_v[pl.ds(p * 16, 16)]
            te1 = te_v[pl.ds(TOK + p * 16, 16)]
            tw1 = tw_v[pl.ds(TOK + p * 16, 16)]
            for e in range(E):
                w = (jnp.where(te0 == e, tw0, zero)
                     + jnp.where(te1 == e, tw1, zero))
                w_v[pl.ds(e * TOK + p * 16, 16)] = w
        pltpu.sync_copy(w_v, w_hbm)


_sc_router = functools.partial(
    pl.kernel,
    mesh=plsc.VectorSubcoreMesh(core_axis_name="c", subcore_axis_name="s"),
    out_type=jax.ShapeDtypeStruct((E * TOK,), jnp.float32),
    scratch_types=[
        pltpu.VMEM((_N_ASSIGN,), jnp.int32),
        pltpu.VMEM((_N_ASSIGN,), jnp.float32),
        pltpu.VMEM((E * TOK,), jnp.float32),
    ],
)(_router_kernel)


def _moe_kernel(x_ref, w_ref, up_ref, gate_ref, down_ref, out_ref, wt_ref):
    e = pl.program_id(0)
    f = pl.program_id(1)
    first = (e == 0) & (f == 0)

    @pl.when(first)
    def _():
        wt_ref[...] = jnp.transpose(w_ref[...])

    xf = x_ref[...]
    up = up_ref[0]
    gate = gate_ref[0]
    down = down_ref[0]

    x1 = jax.lax.dot_general(xf, up, (((1,), (1,)), ((), ())),
                             preferred_element_type=jnp.float32)
    x2 = jax.lax.dot_general(xf, gate, (((1,), (1,)), ((), ())),
                             preferred_element_type=jnp.float32)
    h = x1 * jax.nn.sigmoid(x1) * x2
    partial = jax.lax.dot_general(h, down, (((1,), (1,)), ((), ())),
                                  preferred_element_type=jnp.float32)

    lane = jax.lax.broadcasted_iota(jnp.int32, (TOK, E), 1)
    w = jnp.sum(jnp.where(lane == e, wt_ref[...], 0.0), axis=1, keepdims=True)
    contrib = partial * w

    @pl.when(first)
    def _():
        out_ref[...] = contrib

    @pl.when(jnp.logical_not(first))
    def _():
        out_ref[...] += contrib


def kernel(x, weights, top_weights, top_experts, up_w, gate_w, down_w):
    bsz, q_len, hidden = x.shape
    tok = bsz * q_len
    xf = x.reshape(tok, hidden)

    te_slots = top_experts.astype(jnp.int32).T.reshape(-1)
    tw_slots = top_weights.T.reshape(-1)
    w_dense = _sc_router(te_slots, tw_slots).reshape(E, tok)

    out = pl.pallas_call(
        _moe_kernel,
        grid=(E, NF),
        in_specs=[
            pl.BlockSpec((tok, hidden), lambda e, f: (0, 0)),
            pl.BlockSpec((E, tok), lambda e, f: (0, 0)),
            pl.BlockSpec((1, FB, hidden), lambda e, f: (e, f, 0)),
            pl.BlockSpec((1, FB, hidden), lambda e, f: (e, f, 0)),
            pl.BlockSpec((1, hidden, FB), lambda e, f: (e, 0, f)),
        ],
        out_specs=pl.BlockSpec((tok, hidden), lambda e, f: (0, 0)),
        out_shape=jax.ShapeDtypeStruct((tok, hidden), jnp.float32),
        scratch_shapes=[pltpu.VMEM((tok, E), jnp.float32)],
    )(xf, w_dense, up_w, gate_w, down_w)

    return out.reshape(bsz, q_len, hidden)


# speedup vs baseline: 1.1263x; 1.1263x over previous
import jax
import jax.numpy as jnp
from jax.experimental import pallas as pl

HIDDEN = 1024
FFN = 2048
E = 16
TOPK = 2
FB = 1024
NF = FFN // FB


def _moe_kernel(x_ref, tw_ref, te_ref, up_ref, gate_ref, down_ref, out_ref):
    e = pl.program_id(0)
    f = pl.program_id(1)

    xf = x_ref[...]
    up = up_ref[0]
    gate = gate_ref[0]
    down = down_ref[0]

    x1 = jax.lax.dot_general(xf, up, (((1,), (1,)), ((), ())),
                             preferred_element_type=jnp.float32)
    x2 = jax.lax.dot_general(xf, gate, (((1,), (1,)), ((), ())),
                             preferred_element_type=jnp.float32)
    h = x1 * jax.nn.sigmoid(x1) * x2
    partial = jax.lax.dot_general(h, down, (((1,), (1,)), ((), ())),
                                  preferred_element_type=jnp.float32)

    mask = te_ref[...] == e
    w = jnp.sum(jnp.where(mask, tw_ref[...], 0.0), axis=1, keepdims=True)
    contrib = partial * w

    first = (e == 0) & (f == 0)

    @pl.when(first)
    def _():
        out_ref[...] = contrib

    @pl.when(jnp.logical_not(first))
    def _():
        out_ref[...] += contrib


def kernel(x, weights, top_weights, top_experts, up_w, gate_w, down_w):
    bsz, q_len, hidden = x.shape
    tok = bsz * q_len
    xf = x.reshape(tok, hidden)
    te = top_experts.astype(jnp.int32)

    out = pl.pallas_call(
        _moe_kernel,
        grid=(E, NF),
        in_specs=[
            pl.BlockSpec((tok, hidden), lambda e, f: (0, 0)),
            pl.BlockSpec((tok, TOPK), lambda e, f: (0, 0)),
            pl.BlockSpec((tok, TOPK), lambda e, f: (0, 0)),
            pl.BlockSpec((1, FB, hidden), lambda e, f: (e, f, 0)),
            pl.BlockSpec((1, FB, hidden), lambda e, f: (e, f, 0)),
            pl.BlockSpec((1, hidden, FB), lambda e, f: (e, 0, f)),
        ],
        out_specs=pl.BlockSpec((tok, hidden), lambda e, f: (0, 0)),
        out_shape=jax.ShapeDtypeStruct((tok, hidden), jnp.float32),
    )(xf, top_weights, te, up_w, gate_w, down_w)

    return out.reshape(bsz, q_len, hidden)
